# baseline (speedup 1.0000x reference)
import functools

import jax
import jax.numpy as jnp
from jax import lax
from jax.experimental import pallas as pl
from jax.experimental.pallas import tpu as pltpu
from jax.experimental.pallas import tpu_sc as plsc

_L = 2
_H = 128
_N = 10000
_E = 320000

_NC = 2
_NS = 16
_NW = _NC * _NS
_PERW = _E // _NW
_KM = 40
_CHM = _PERW // _KM
_KP = 40
_CHP = _PERW // _KP
_NPT = 624
_ZCH = 48

_BN = 2000
_BE = 4000


def _sc_msg(e, x, src4, dst4):
  @functools.partial(
      pl.kernel,
      out_type=jax.ShapeDtypeStruct((_NC, _N, _H), jnp.float32),
      mesh=plsc.VectorSubcoreMesh(
          core_axis_name="c", subcore_axis_name="s",
          num_cores=_NC, num_subcores=_NS),
      scratch_types=[
          pltpu.VMEM_SHARED((_N, _H), jnp.float32),
          pltpu.VMEM((1, _KM), jnp.int32), pltpu.VMEM((1, _KM), jnp.int32),
          pltpu.VMEM((1, _KM), jnp.int32),
          pltpu.VMEM((1, _KM), jnp.int32), pltpu.VMEM((1, _KM), jnp.int32),
          pltpu.VMEM((1, _KM), jnp.int32),
          pltpu.VMEM((_KM, _H), jnp.float32), pltpu.VMEM((_KM, _H), jnp.float32),
          pltpu.VMEM((_KM, _H), jnp.float32),
          pltpu.VMEM((_KM, _H), jnp.float32), pltpu.VMEM((_KM, _H), jnp.float32),
          pltpu.VMEM((_KM, _H), jnp.float32),
          pltpu.VMEM((_ZCH, _H), jnp.float32),
          pltpu.SemaphoreType.DMA, pltpu.SemaphoreType.DMA,
          pltpu.SemaphoreType.DMA, pltpu.SemaphoreType.DMA,
          pltpu.SemaphoreType.DMA, pltpu.SemaphoreType.DMA,
          pltpu.SemaphoreType.DMA, pltpu.SemaphoreType.DMA,
          pltpu.SemaphoreType.DMA, pltpu.SemaphoreType.DMA,
          pltpu.SemaphoreType.DMA, pltpu.SemaphoreType.DMA,
          pltpu.SemaphoreType.DMA, pltpu.SemaphoreType.DMA,
          pltpu.SemaphoreType.DMA,
      ],
  )
  def body(e_hbm, x_hbm, src_hbm, dst_hbm, out_hbm,
           acc, sv0, sv1, sv2, dv0, dv1, dv2, eb0, eb1, eb2,
           xb0, xb1, xb2, zbuf,
           si0, si1, si2, se0, se1, se2, sx0, sx1, sx2, ss0, ss1, ss2,
           sz, sw0, sw1):
    sv = (sv0, sv1, sv2); dv = (dv0, dv1, dv2)
    eb = (eb0, eb1, eb2); xb = (xb0, xb1, xb2)
    si = (si0, si1, si2); se = (se0, se1, se2)
    sx = (sx0, sx1, sx2); ss = (ss0, ss1, ss2)
    c = lax.axis_index("c")
    s = lax.axis_index("s")
    wid = c * _NS + s
    base = wid * _PERW

    def issue_idx(j, b):
      pltpu.async_copy(src_hbm.at[wid, j], sv[b], si[b])
      pltpu.async_copy(dst_hbm.at[wid, j], dv[b], si[b])

    def wait_idx(b):
      pltpu.make_async_copy(src_hbm.at[wid, 0], sv[b], si[b]).wait()
      pltpu.make_async_copy(dst_hbm.at[wid, 0], dv[b], si[b]).wait()

    def issue_load(j, b):
      pltpu.async_copy(e_hbm.at[pl.ds(base + j * _KM, _KM)], eb[b], se[b])
      pltpu.async_copy(x_hbm.at[sv[b].at[0]], xb[b], sx[b])

    def wait_load(b):
      pltpu.make_async_copy(e_hbm.at[pl.ds(0, _KM)], eb[b], se[b]).wait()
      pltpu.make_async_copy(x_hbm.at[sv[b].at[0]], xb[b], sx[b]).wait()

    def wait_scatter(b):
      pltpu.make_async_copy(xb[b], acc.at[dv[b].at[0]], ss[b]).wait()

    def zrow(r, _):
      for j in range(_H // 16):
        zbuf[r, pl.ds(j * 16, 16)] = jnp.zeros((16,), jnp.float32)
      return 0
    lax.fori_loop(0, _ZCH, zrow, 0)
    row0 = s * _NPT
    for t in range(_NPT // _ZCH):
      pltpu.async_copy(zbuf, acc.at[pl.ds(row0 + t * _ZCH, _ZCH)], sz)

    @pl.when(s == _NS - 1)
    def _():
      pltpu.sync_copy(zbuf.at[pl.ds(0, 16)], acc.at[pl.ds(_NS * _NPT, 16)])
    for t in range(_NPT // _ZCH):
      pltpu.make_async_copy(zbuf, acc.at[pl.ds(row0, _ZCH)], sz).wait()
    plsc.subcore_barrier()

    issue_idx(0, 0)
    issue_idx(1, 1)
    issue_idx(2, 2)
    wait_idx(0)
    issue_load(0, 0)
    wait_idx(1)
    issue_load(1, 1)

    def do_chunk(j, b, n):
      @pl.when(j + 2 < _CHM)
      def _():
        wait_idx(n)

      @pl.when(jnp.logical_and(j >= 1, j + 2 < _CHM))
      def _():
        wait_scatter(n)

      @pl.when(j + 2 < _CHM)
      def _():
        issue_load(j + 2, n)

      wait_load(b)

      ebb, xbb = eb[b], xb[b]

      @plsc.parallel_loop(0, _KM, unroll=4)
      def _(r):
        for jj in range(_H // 16):
          sl = pl.ds(jj * 16, 16)
          xbb[r, sl] = jnp.maximum(xbb[r, sl] + ebb[r, sl], 0.0)

      pltpu.async_copy(xb[b], acc.at[dv[b].at[0]], ss[b], add=True)

      @pl.when(j + 3 < _CHM)
      def _():
        issue_idx(j + 3, b)

    def triple(i, _):
      for b in range(3):
        do_chunk(3 * i + b, b, (b + 2) % 3)
      return 0
    lax.fori_loop(0, _CHM // 3, triple, 0)
    for j in range(_CHM - _CHM % 3, _CHM):
      do_chunk(jnp.int32(j), j % 3, (j + 2) % 3)
    wait_scatter((_CHM - 3) % 3)
    wait_scatter((_CHM - 2) % 3)
    wait_scatter((_CHM - 1) % 3)
    plsc.subcore_barrier()

    sw = (sw0, sw1)
    _WB = 24
    for t in range(_NPT // _WB):
      h = pl.ds((t % 2) * _WB, _WB)
      sl = pl.ds(row0 + t * _WB, _WB)
      if t >= 2:
        pltpu.make_async_copy(zbuf.at[h], out_hbm.at[c, pl.ds(0, _WB)],
                              sw[t % 2]).wait()
      pltpu.sync_copy(acc.at[sl], zbuf.at[h])
      pltpu.async_copy(zbuf.at[h], out_hbm.at[c, sl], sw[t % 2])
    for b2 in range(2):
      pltpu.make_async_copy(zbuf.at[pl.ds(b2 * _WB, _WB)],
                            out_hbm.at[c, pl.ds(0, _WB)], sw[b2]).wait()

    @pl.when(s == _NS - 1)
    def _():
      tsl = pl.ds(_NS * _NPT, 16)
      pltpu.sync_copy(acc.at[tsl], zbuf.at[pl.ds(0, 16)])
      pltpu.sync_copy(zbuf.at[pl.ds(0, 16)], out_hbm.at[c, tsl])

  return body(e, x, src4, dst4)


def _sc_pair(a, b, src3, dst3):
  @functools.partial(
      pl.kernel,
      out_type=jax.ShapeDtypeStruct((_E, _H), jnp.float32),
      mesh=plsc.VectorSubcoreMesh(
          core_axis_name="c", subcore_axis_name="s",
          num_cores=_NC, num_subcores=_NS),
      scratch_types=[
          pltpu.VMEM((_CHP, _KP), jnp.int32),
          pltpu.VMEM((_CHP, _KP), jnp.int32),
          pltpu.VMEM((_KP, _H), jnp.float32), pltpu.VMEM((_KP, _H), jnp.float32),
          pltpu.VMEM((_KP, _H), jnp.float32),
          pltpu.VMEM((_KP, _H), jnp.float32), pltpu.VMEM((_KP, _H), jnp.float32),
          pltpu.VMEM((_KP, _H), jnp.float32),
          pltpu.VMEM((_KP, _H), jnp.float32), pltpu.VMEM((_KP, _H), jnp.float32),
          pltpu.VMEM((_KP, _H), jnp.float32),
          pltpu.SemaphoreType.DMA, pltpu.SemaphoreType.DMA,
          pltpu.SemaphoreType.DMA, pltpu.SemaphoreType.DMA,
          pltpu.SemaphoreType.DMA, pltpu.SemaphoreType.DMA,
          pltpu.SemaphoreType.DMA, pltpu.SemaphoreType.DMA,
          pltpu.SemaphoreType.DMA,
      ],
  )
  def body(a_hbm, b_hbm, src_hbm, dst_hbm, out_hbm, sv, dv,
           ab0, ab1, ab2, bb0, bb1, bb2, ob0, ob1, ob2,
           sa0, sa1, sa2, sb0, sb1, sb2, so0, so1, so2):
    ab = (ab0, ab1, ab2); bb = (bb0, bb1, bb2); ob = (ob0, ob1, ob2)
    sa = (sa0, sa1, sa2); sb = (sb0, sb1, sb2); so = (so0, so1, so2)
    c = lax.axis_index("c")
    s = lax.axis_index("s")
    wid = c * _NS + s
    base = wid * _PERW

    pltpu.sync_copy(src_hbm.at[wid], sv)
    pltpu.sync_copy(dst_hbm.at[wid], dv)

    def issue_g(j, b):
      pltpu.async_copy(a_hbm.at[sv.at[j]], ab[b], sa[b])
      pltpu.async_copy(b_hbm.at[dv.at[j]], bb[b], sb[b])

    def wait_g(b):
      pltpu.make_async_copy(a_hbm.at[sv.at[0]], ab[b], sa[b]).wait()
      pltpu.make_async_copy(b_hbm.at[dv.at[0]], bb[b], sb[b]).wait()

    def wait_out(b):
      pltpu.make_async_copy(ob[b], out_hbm.at[pl.ds(0, _KP)], so[b]).wait()

    issue_g(0, 0)
    issue_g(1, 1)
    issue_g(2, 2)

    def do_chunk(j, b):
      wait_g(b)

      @pl.when(j >= 3)
      def _():
        wait_out(b)

      abb, bbb, obb = ab[b], bb[b], ob[b]

      @plsc.parallel_loop(0, _KP, unroll=4)
      def _(r):
        for jj in range(_H // 16):
          sl = pl.ds(jj * 16, 16)
          obb[r, sl] = abb[r, sl] + bbb[r, sl]

      @pl.when(j + 3 < _CHP)
      def _():
        issue_g(j + 3, b)
      pltpu.async_copy(ob[b], out_hbm.at[pl.ds(base + j * _KP, _KP)], so[b])

    def triple(i, _):
      for b in range(3):
        do_chunk(3 * i + b, b)
      return 0
    lax.fori_loop(0, _CHP // 3, triple, 0)
    for j in range(_CHP - _CHP % 3, _CHP):
      do_chunk(jnp.int32(j), j % 3)
    wait_out((_CHP - 3) % 3)
    wait_out((_CHP - 2) % 3)
    wait_out((_CHP - 1) % 3)

  return body(a, b, src3, dst3)


def _full(shape):
  return pl.BlockSpec(shape, lambda i: (0, 0))


def _tc_edge_mm(ea, w, bias):
  def body(ea_ref, w_ref, b_ref, o_ref):
    o_ref[...] = (jnp.dot(ea_ref[...], w_ref[...],
                          preferred_element_type=jnp.float32) + b_ref[...])
  return pl.pallas_call(
      body,
      grid=(_E // _BE,),
      in_specs=[pl.BlockSpec((_BE, _H), lambda i: (i, 0)),
                _full((_H, _H)), _full((1, _H))],
      out_specs=pl.BlockSpec((_BE, _H), lambda i: (i, 0)),
      out_shape=jax.ShapeDtypeStruct((_E, _H), jnp.float32),
  )(ea, w, bias)


def _tc_stats(x, parts, w1, b1, w2, b2):
  def body(x_ref, p0_ref, p1_ref, w1_ref, b1_ref, w2_ref, b2_ref,
           h2_ref, ssum_ref, ssq_ref):
    h = x_ref[...] + p0_ref[0] + p1_ref[0]
    h1 = jnp.maximum(jnp.dot(h, w1_ref[...],
                             preferred_element_type=jnp.float32) + b1_ref[...],
                     0.0)
    h2 = jnp.dot(h1, w2_ref[...],
                 preferred_element_type=jnp.float32) + b2_ref[...]
    h2_ref[...] = h2

    @pl.when(pl.program_id(0) == 0)
    def _():
      ssum_ref[...] = jnp.zeros_like(ssum_ref)
      ssq_ref[...] = jnp.zeros_like(ssq_ref)
    ssum_ref[...] += jnp.sum(h2, axis=0, keepdims=True)
    ssq_ref[...] += jnp.sum(h2 * h2, axis=0, keepdims=True)

  return pl.pallas_call(
      body,
      grid=(_N // _BN,),
      in_specs=[pl.BlockSpec((_BN, _H), lambda i: (i, 0)),
                pl.BlockSpec((1, _BN, _H), lambda i: (0, i, 0)),
                pl.BlockSpec((1, _BN, _H), lambda i: (1, i, 0)),
                _full((_H, _H)), _full((1, _H)),
                _full((_H, _H)), _full((1, _H))],
      out_specs=[pl.BlockSpec((_BN, _H), lambda i: (i, 0)),
                 _full((1, _H)), _full((1, _H))],
      out_shape=[jax.ShapeDtypeStruct((_N, _H), jnp.float32),
                 jax.ShapeDtypeStruct((1, _H), jnp.float32),
                 jax.ShapeDtypeStruct((1, _H), jnp.float32)],
  )(x, parts, parts, w1, b1, w2, b2)


def _tc_fin(x, h2, ssum, ssq, g, bta, ua, ub):
  def body(x_ref, h2_ref, ssum_ref, ssq_ref, g_ref, b_ref, ua_ref, ub_ref,
           xn_ref, a_ref, bo_ref):
    mu = ssum_ref[...] * (1.0 / _N)
    var = ssq_ref[...] * (1.0 / _N) - mu * mu
    inv = lax.rsqrt(var + 1e-5)
    xb = g_ref[...] * (h2_ref[...] - mu) * inv + b_ref[...]
    xn = (x_ref[...] + jnp.maximum(xb, 0.0)) * 0.5
    xn_ref[...] = xn
    a_ref[...] = jnp.dot(xn, ua_ref[...], preferred_element_type=jnp.float32)
    bo_ref[...] = jnp.dot(xn, ub_ref[...], preferred_element_type=jnp.float32)

  return pl.pallas_call(
      body,
      grid=(_N // _BN,),
      in_specs=[pl.BlockSpec((_BN, _H), lambda i: (i, 0)),
                pl.BlockSpec((_BN, _H), lambda i: (i, 0)),
                _full((1, _H)), _full((1, _H)),
                _full((1, _H)), _full((1, _H)),
                _full((_H, _H)), _full((_H, _H))],
      out_specs=[pl.BlockSpec((_BN, _H), lambda i: (i, 0)),
                 pl.BlockSpec((_BN, _H), lambda i: (i, 0)),
                 pl.BlockSpec((_BN, _H), lambda i: (i, 0))],
      out_shape=[jax.ShapeDtypeStruct((_N, _H), jnp.float32),
                 jax.ShapeDtypeStruct((_N, _H), jnp.float32),
                 jax.ShapeDtypeStruct((_N, _H), jnp.float32)],
  )(x, h2, ssum, ssq, g, bta, ua, ub)


def _tc_upd(ea, tpre, uc, c1, u2, c2, wn=None, bn=None):
  has_next = wn is not None

  def body(ea_ref, t_ref, uc_ref, c1_ref, u2_ref, c2_ref, *rest):
    if has_next:
      wn_ref, bn_ref, ean_ref, en_ref = rest
    else:
      (ean_ref,) = rest
    cmsg = jnp.dot(ea_ref[...], uc_ref[...],
                   preferred_element_type=jnp.float32) + c1_ref[...]
    u = jnp.maximum(t_ref[...] + cmsg, 0.0)
    upd = jnp.dot(u, u2_ref[...],
                  preferred_element_type=jnp.float32) + c2_ref[...]
    ean = ea_ref[...] + 0.5 * upd
    ean_ref[...] = ean
    if has_next:
      en_ref[...] = jnp.dot(ean, wn_ref[...],
                            preferred_element_type=jnp.float32) + bn_ref[...]

  in_specs = [pl.BlockSpec((_BE, _H), lambda i: (i, 0)),
              pl.BlockSpec((_BE, _H), lambda i: (i, 0)),
              _full((_H, _H)), _full((1, _H)),
              _full((_H, _H)), _full((1, _H))]
  args = [ea, tpre, uc, c1, u2, c2]
  out_specs = [pl.BlockSpec((_BE, _H), lambda i: (i, 0))]
  out_shape = [jax.ShapeDtypeStruct((_E, _H), jnp.float32)]
  if has_next:
    in_specs += [_full((_H, _H)), _full((1, _H))]
    args += [wn, bn]
    out_specs.append(pl.BlockSpec((_BE, _H), lambda i: (i, 0)))
    out_shape.append(jax.ShapeDtypeStruct((_E, _H), jnp.float32))

  res = pl.pallas_call(
      body,
      grid=(_E // _BE,),
      in_specs=in_specs,
      out_specs=out_specs,
      out_shape=out_shape,
  )(*args)
  return res if has_next else res[0]


def kernel(x, edge_attr, edge_index, We, be, W1, b1, W2, b2,
           gamma, beta, Ue1, ce1, Ue2, ce2):
  src3 = edge_index[0].reshape(_NW, _CHP, _KP)
  dst3 = edge_index[1].reshape(_NW, _CHP, _KP)
  src4 = edge_index[0].reshape(_NW, _CHM, 1, _KM)
  dst4 = edge_index[1].reshape(_NW, _CHM, 1, _KM)

  ea = edge_attr
  e = _tc_edge_mm(ea, We[0], be[0].reshape(1, _H))
  for i in range(_L):
    parts = _sc_msg(e, x, src4, dst4)
    h2, ssum, ssq = _tc_stats(x, parts,
                              W1[i], b1[i].reshape(1, _H),
                              W2[i], b2[i].reshape(1, _H))
    x, a, bgath = _tc_fin(x, h2, ssum, ssq,
                          gamma[i].reshape(1, _H), beta[i].reshape(1, _H),
                          Ue1[i, :_H], Ue1[i, _H:2 * _H])
    tpre = _sc_pair(a, bgath, src3, dst3)
    if i + 1 < _L:
      ea, e = _tc_upd(ea, tpre, Ue1[i, 2 * _H:], ce1[i].reshape(1, _H),
                      Ue2[i], ce2[i].reshape(1, _H),
                      We[i + 1], be[i + 1].reshape(1, _H))
    else:
      ea = _tc_upd(ea, tpre, Ue1[i, 2 * _H:], ce1[i].reshape(1, _H),
                   Ue2[i], ce2[i].reshape(1, _H))
  return x, ea


# speedup vs baseline: 1.0083x; 1.0083x over previous
import functools

import jax
import jax.numpy as jnp
from jax import lax
from jax.experimental import pallas as pl
from jax.experimental.pallas import tpu as pltpu
from jax.experimental.pallas import tpu_sc as plsc

_L = 2
_H = 128
_N = 10000
_E = 320000

_NC = 2
_NS = 16
_NW = _NC * _NS
_PERW = _E // _NW
_KM = 40
_CHM = _PERW // _KM
_KP = 80
_CHP = _PERW // _KP
_NPT = 624
_ZCH = 48

_BN = 2000
_BE = 4000


def _sc_msg(e, x, src4, dst4):
  @functools.partial(
      pl.kernel,
      out_type=jax.ShapeDtypeStruct((_NC, _N, _H), jnp.float32),
      mesh=plsc.VectorSubcoreMesh(
          core_axis_name="c", subcore_axis_name="s",
          num_cores=_NC, num_subcores=_NS),
      scratch_types=[
          pltpu.VMEM_SHARED((_N, _H), jnp.float32),
          pltpu.VMEM((1, _KM), jnp.int32), pltpu.VMEM((1, _KM), jnp.int32),
          pltpu.VMEM((1, _KM), jnp.int32),
          pltpu.VMEM((1, _KM), jnp.int32), pltpu.VMEM((1, _KM), jnp.int32),
          pltpu.VMEM((1, _KM), jnp.int32),
          pltpu.VMEM((_KM, _H), jnp.float32), pltpu.VMEM((_KM, _H), jnp.float32),
          pltpu.VMEM((_KM, _H), jnp.float32),
          pltpu.VMEM((_KM, _H), jnp.float32), pltpu.VMEM((_KM, _H), jnp.float32),
          pltpu.VMEM((_KM, _H), jnp.float32),
          pltpu.VMEM((_ZCH, _H), jnp.float32),
          pltpu.SemaphoreType.DMA, pltpu.SemaphoreType.DMA,
          pltpu.SemaphoreType.DMA, pltpu.SemaphoreType.DMA,
          pltpu.SemaphoreType.DMA, pltpu.SemaphoreType.DMA,
          pltpu.SemaphoreType.DMA, pltpu.SemaphoreType.DMA,
          pltpu.SemaphoreType.DMA, pltpu.SemaphoreType.DMA,
          pltpu.SemaphoreType.DMA, pltpu.SemaphoreType.DMA,
          pltpu.SemaphoreType.DMA, pltpu.SemaphoreType.DMA,
          pltpu.SemaphoreType.DMA,
      ],
  )
  def body(e_hbm, x_hbm, src_hbm, dst_hbm, out_hbm,
           acc, sv0, sv1, sv2, dv0, dv1, dv2, eb0, eb1, eb2,
           xb0, xb1, xb2, zbuf,
           si0, si1, si2, se0, se1, se2, sx0, sx1, sx2, ss0, ss1, ss2,
           sz, sw0, sw1):
    sv = (sv0, sv1, sv2); dv = (dv0, dv1, dv2)
    eb = (eb0, eb1, eb2); xb = (xb0, xb1, xb2)
    si = (si0, si1, si2); se = (se0, se1, se2)
    sx = (sx0, sx1, sx2); ss = (ss0, ss1, ss2)
    c = lax.axis_index("c")
    s = lax.axis_index("s")
    wid = c * _NS + s
    base = wid * _PERW

    def issue_idx(j, b):
      pltpu.async_copy(src_hbm.at[wid, j], sv[b], si[b])
      pltpu.async_copy(dst_hbm.at[wid, j], dv[b], si[b])

    def wait_idx(b):
      pltpu.make_async_copy(src_hbm.at[wid, 0], sv[b], si[b]).wait()
      pltpu.make_async_copy(dst_hbm.at[wid, 0], dv[b], si[b]).wait()

    def issue_load(j, b):
      pltpu.async_copy(e_hbm.at[pl.ds(base + j * _KM, _KM)], eb[b], se[b])
      pltpu.async_copy(x_hbm.at[sv[b].at[0]], xb[b], sx[b])

    def wait_load(b):
      pltpu.make_async_copy(e_hbm.at[pl.ds(0, _KM)], eb[b], se[b]).wait()
      pltpu.make_async_copy(x_hbm.at[sv[b].at[0]], xb[b], sx[b]).wait()

    def wait_scatter(b):
      pltpu.make_async_copy(xb[b], acc.at[dv[b].at[0]], ss[b]).wait()

    def zrow(r, _):
      for j in range(_H // 16):
        zbuf[r, pl.ds(j * 16, 16)] = jnp.zeros((16,), jnp.float32)
      return 0
    lax.fori_loop(0, _ZCH, zrow, 0)
    row0 = s * _NPT
    for t in range(_NPT // _ZCH):
      pltpu.async_copy(zbuf, acc.at[pl.ds(row0 + t * _ZCH, _ZCH)], sz)

    @pl.when(s == _NS - 1)
    def _():
      pltpu.sync_copy(zbuf.at[pl.ds(0, 16)], acc.at[pl.ds(_NS * _NPT, 16)])
    for t in range(_NPT // _ZCH):
      pltpu.make_async_copy(zbuf, acc.at[pl.ds(row0, _ZCH)], sz).wait()
    plsc.subcore_barrier()

    issue_idx(0, 0)
    issue_idx(1, 1)
    issue_idx(2, 2)
    wait_idx(0)
    issue_load(0, 0)
    wait_idx(1)
    issue_load(1, 1)

    def do_chunk(j, b, n):
      @pl.when(j + 2 < _CHM)
      def _():
        wait_idx(n)

      @pl.when(jnp.logical_and(j >= 1, j + 2 < _CHM))
      def _():
        wait_scatter(n)

      @pl.when(j + 2 < _CHM)
      def _():
        issue_load(j + 2, n)

      wait_load(b)

      ebb, xbb = eb[b], xb[b]

      @plsc.parallel_loop(0, _KM, unroll=4)
      def _(r):
        for jj in range(_H // 16):
          sl = pl.ds(jj * 16, 16)
          xbb[r, sl] = jnp.maximum(xbb[r, sl] + ebb[r, sl], 0.0)

      pltpu.async_copy(xb[b], acc.at[dv[b].at[0]], ss[b], add=True)

      @pl.when(j + 3 < _CHM)
      def _():
        issue_idx(j + 3, b)

    def triple(i, _):
      for b in range(3):
        do_chunk(3 * i + b, b, (b + 2) % 3)
      return 0
    lax.fori_loop(0, _CHM // 3, triple, 0)
    for j in range(_CHM - _CHM % 3, _CHM):
      do_chunk(jnp.int32(j), j % 3, (j + 2) % 3)
    wait_scatter((_CHM - 3) % 3)
    wait_scatter((_CHM - 2) % 3)
    wait_scatter((_CHM - 1) % 3)
    plsc.subcore_barrier()

    sw = (sw0, sw1)
    _WB = 24
    for t in range(_NPT // _WB):
      h = pl.ds((t % 2) * _WB, _WB)
      sl = pl.ds(row0 + t * _WB, _WB)
      if t >= 2:
        pltpu.make_async_copy(zbuf.at[h], out_hbm.at[c, pl.ds(0, _WB)],
                              sw[t % 2]).wait()
      pltpu.sync_copy(acc.at[sl], zbuf.at[h])
      pltpu.async_copy(zbuf.at[h], out_hbm.at[c, sl], sw[t % 2])
    for b2 in range(2):
      pltpu.make_async_copy(zbuf.at[pl.ds(b2 * _WB, _WB)],
                            out_hbm.at[c, pl.ds(0, _WB)], sw[b2]).wait()

    @pl.when(s == _NS - 1)
    def _():
      tsl = pl.ds(_NS * _NPT, 16)
      pltpu.sync_copy(acc.at[tsl], zbuf.at[pl.ds(0, 16)])
      pltpu.sync_copy(zbuf.at[pl.ds(0, 16)], out_hbm.at[c, tsl])

  return body(e, x, src4, dst4)


def _sc_pair(a, b, src3, dst3):
  @functools.partial(
      pl.kernel,
      out_type=jax.ShapeDtypeStruct((_E, _H), jnp.float32),
      mesh=plsc.VectorSubcoreMesh(
          core_axis_name="c", subcore_axis_name="s",
          num_cores=_NC, num_subcores=_NS),
      scratch_types=[
          pltpu.VMEM((_CHP, _KP), jnp.int32),
          pltpu.VMEM((_CHP, _KP), jnp.int32),
          pltpu.VMEM((_KP, _H), jnp.float32), pltpu.VMEM((_KP, _H), jnp.float32),
          pltpu.VMEM((_KP, _H), jnp.float32),
          pltpu.VMEM((_KP, _H), jnp.float32), pltpu.VMEM((_KP, _H), jnp.float32),
          pltpu.VMEM((_KP, _H), jnp.float32),
          pltpu.VMEM((_KP, _H), jnp.float32), pltpu.VMEM((_KP, _H), jnp.float32),
          pltpu.VMEM((_KP, _H), jnp.float32),
          pltpu.SemaphoreType.DMA, pltpu.SemaphoreType.DMA,
          pltpu.SemaphoreType.DMA, pltpu.SemaphoreType.DMA,
          pltpu.SemaphoreType.DMA, pltpu.SemaphoreType.DMA,
          pltpu.SemaphoreType.DMA, pltpu.SemaphoreType.DMA,
          pltpu.SemaphoreType.DMA,
      ],
  )
  def body(a_hbm, b_hbm, src_hbm, dst_hbm, out_hbm, sv, dv,
           ab0, ab1, ab2, bb0, bb1, bb2, ob0, ob1, ob2,
           sa0, sa1, sa2, sb0, sb1, sb2, so0, so1, so2):
    ab = (ab0, ab1, ab2); bb = (bb0, bb1, bb2); ob = (ob0, ob1, ob2)
    sa = (sa0, sa1, sa2); sb = (sb0, sb1, sb2); so = (so0, so1, so2)
    c = lax.axis_index("c")
    s = lax.axis_index("s")
    wid = c * _NS + s
    base = wid * _PERW

    pltpu.sync_copy(src_hbm.at[wid], sv)
    pltpu.sync_copy(dst_hbm.at[wid], dv)

    def issue_g(j, b):
      pltpu.async_copy(a_hbm.at[sv.at[j]], ab[b], sa[b])
      pltpu.async_copy(b_hbm.at[dv.at[j]], bb[b], sb[b])

    def wait_g(b):
      pltpu.make_async_copy(a_hbm.at[sv.at[0]], ab[b], sa[b]).wait()
      pltpu.make_async_copy(b_hbm.at[dv.at[0]], bb[b], sb[b]).wait()

    def wait_out(b):
      pltpu.make_async_copy(ob[b], out_hbm.at[pl.ds(0, _KP)], so[b]).wait()

    issue_g(0, 0)
    issue_g(1, 1)
    issue_g(2, 2)

    def do_chunk(j, b):
      wait_g(b)

      @pl.when(j >= 3)
      def _():
        wait_out(b)

      abb, bbb, obb = ab[b], bb[b], ob[b]

      @plsc.parallel_loop(0, _KP, unroll=4)
      def _(r):
        for jj in range(_H // 16):
          sl = pl.ds(jj * 16, 16)
          obb[r, sl] = abb[r, sl] + bbb[r, sl]

      @pl.when(j + 3 < _CHP)
      def _():
        issue_g(j + 3, b)
      pltpu.async_copy(ob[b], out_hbm.at[pl.ds(base + j * _KP, _KP)], so[b])

    def triple(i, _):
      for b in range(3):
        do_chunk(3 * i + b, b)
      return 0
    lax.fori_loop(0, _CHP // 3, triple, 0)
    for j in range(_CHP - _CHP % 3, _CHP):
      do_chunk(jnp.int32(j), j % 3)
    wait_out((_CHP - 3) % 3)
    wait_out((_CHP - 2) % 3)
    wait_out((_CHP - 1) % 3)

  return body(a, b, src3, dst3)


def _full(shape):
  return pl.BlockSpec(shape, lambda i: (0, 0))


def _tc_edge_mm(ea, w, bias):
  def body(ea_ref, w_ref, b_ref, o_ref):
    o_ref[...] = (jnp.dot(ea_ref[...], w_ref[...],
                          preferred_element_type=jnp.float32) + b_ref[...])
  return pl.pallas_call(
      body,
      grid=(_E // _BE,),
      in_specs=[pl.BlockSpec((_BE, _H), lambda i: (i, 0)),
                _full((_H, _H)), _full((1, _H))],
      out_specs=pl.BlockSpec((_BE, _H), lambda i: (i, 0)),
      out_shape=jax.ShapeDtypeStruct((_E, _H), jnp.float32),
  )(ea, w, bias)


def _tc_stats(x, parts, w1, b1, w2, b2):
  def body(x_ref, p0_ref, p1_ref, w1_ref, b1_ref, w2_ref, b2_ref,
           h2_ref, ssum_ref, ssq_ref):
    h = x_ref[...] + p0_ref[0] + p1_ref[0]
    h1 = jnp.maximum(jnp.dot(h, w1_ref[...],
                             preferred_element_type=jnp.float32) + b1_ref[...],
                     0.0)
    h2 = jnp.dot(h1, w2_ref[...],
                 preferred_element_type=jnp.float32) + b2_ref[...]
    h2_ref[...] = h2

    @pl.when(pl.program_id(0) == 0)
    def _():
      ssum_ref[...] = jnp.zeros_like(ssum_ref)
      ssq_ref[...] = jnp.zeros_like(ssq_ref)
    ssum_ref[...] += jnp.sum(h2, axis=0, keepdims=True)
    ssq_ref[...] += jnp.sum(h2 * h2, axis=0, keepdims=True)

  return pl.pallas_call(
      body,
      grid=(_N // _BN,),
      in_specs=[pl.BlockSpec((_BN, _H), lambda i: (i, 0)),
                pl.BlockSpec((1, _BN, _H), lambda i: (0, i, 0)),
                pl.BlockSpec((1, _BN, _H), lambda i: (1, i, 0)),
                _full((_H, _H)), _full((1, _H)),
                _full((_H, _H)), _full((1, _H))],
      out_specs=[pl.BlockSpec((_BN, _H), lambda i: (i, 0)),
                 _full((1, _H)), _full((1, _H))],
      out_shape=[jax.ShapeDtypeStruct((_N, _H), jnp.float32),
                 jax.ShapeDtypeStruct((1, _H), jnp.float32),
                 jax.ShapeDtypeStruct((1, _H), jnp.float32)],
  )(x, parts, parts, w1, b1, w2, b2)


def _tc_fin(x, h2, ssum, ssq, g, bta, ua, ub):
  def body(x_ref, h2_ref, ssum_ref, ssq_ref, g_ref, b_ref, ua_ref, ub_ref,
           xn_ref, a_ref, bo_ref):
    mu = ssum_ref[...] * (1.0 / _N)
    var = ssq_ref[...] * (1.0 / _N) - mu * mu
    inv = lax.rsqrt(var + 1e-5)
    xb = g_ref[...] * (h2_ref[...] - mu) * inv + b_ref[...]
    xn = (x_ref[...] + jnp.maximum(xb, 0.0)) * 0.5
    xn_ref[...] = xn
    a_ref[...] = jnp.dot(xn, ua_ref[...], preferred_element_type=jnp.float32)
    bo_ref[...] = jnp.dot(xn, ub_ref[...], preferred_element_type=jnp.float32)

  return pl.pallas_call(
      body,
      grid=(_N // _BN,),
      in_specs=[pl.BlockSpec((_BN, _H), lambda i: (i, 0)),
                pl.BlockSpec((_BN, _H), lambda i: (i, 0)),
                _full((1, _H)), _full((1, _H)),
                _full((1, _H)), _full((1, _H)),
                _full((_H, _H)), _full((_H, _H))],
      out_specs=[pl.BlockSpec((_BN, _H), lambda i: (i, 0)),
                 pl.BlockSpec((_BN, _H), lambda i: (i, 0)),
                 pl.BlockSpec((_BN, _H), lambda i: (i, 0))],
      out_shape=[jax.ShapeDtypeStruct((_N, _H), jnp.float32),
                 jax.ShapeDtypeStruct((_N, _H), jnp.float32),
                 jax.ShapeDtypeStruct((_N, _H), jnp.float32)],
  )(x, h2, ssum, ssq, g, bta, ua, ub)


def _tc_upd(ea, tpre, uc, c1, u2, c2, wn=None, bn=None):
  has_next = wn is not None

  def body(ea_ref, t_ref, uc_ref, c1_ref, u2_ref, c2_ref, *rest):
    if has_next:
      wn_ref, bn_ref, ean_ref, en_ref = rest
    else:
      (ean_ref,) = rest
    cmsg = jnp.dot(ea_ref[...], uc_ref[...],
                   preferred_element_type=jnp.float32) + c1_ref[...]
    u = jnp.maximum(t_ref[...] + cmsg, 0.0)
    upd = jnp.dot(u, u2_ref[...],
                  preferred_element_type=jnp.float32) + c2_ref[...]
    ean = ea_ref[...] + 0.5 * upd
    ean_ref[...] = ean
    if has_next:
      en_ref[...] = jnp.dot(ean, wn_ref[...],
                            preferred_element_type=jnp.float32) + bn_ref[...]

  in_specs = [pl.BlockSpec((_BE, _H), lambda i: (i, 0)),
              pl.BlockSpec((_BE, _H), lambda i: (i, 0)),
              _full((_H, _H)), _full((1, _H)),
              _full((_H, _H)), _full((1, _H))]
  args = [ea, tpre, uc, c1, u2, c2]
  out_specs = [pl.BlockSpec((_BE, _H), lambda i: (i, 0))]
  out_shape = [jax.ShapeDtypeStruct((_E, _H), jnp.float32)]
  if has_next:
    in_specs += [_full((_H, _H)), _full((1, _H))]
    args += [wn, bn]
    out_specs.append(pl.BlockSpec((_BE, _H), lambda i: (i, 0)))
    out_shape.append(jax.ShapeDtypeStruct((_E, _H), jnp.float32))

  res = pl.pallas_call(
      body,
      grid=(_E // _BE,),
      in_specs=in_specs,
      out_specs=out_specs,
      out_shape=out_shape,
  )(*args)
  return res if has_next else res[0]


def kernel(x, edge_attr, edge_index, We, be, W1, b1, W2, b2,
           gamma, beta, Ue1, ce1, Ue2, ce2):
  src3 = edge_index[0].reshape(_NW, _CHP, _KP)
  dst3 = edge_index[1].reshape(_NW, _CHP, _KP)
  src4 = edge_index[0].reshape(_NW, _CHM, 1, _KM)
  dst4 = edge_index[1].reshape(_NW, _CHM, 1, _KM)

  ea = edge_attr
  e = _tc_edge_mm(ea, We[0], be[0].reshape(1, _H))
  for i in range(_L):
    parts = _sc_msg(e, x, src4, dst4)
    h2, ssum, ssq = _tc_stats(x, parts,
                              W1[i], b1[i].reshape(1, _H),
                              W2[i], b2[i].reshape(1, _H))
    x, a, bgath = _tc_fin(x, h2, ssum, ssq,
                          gamma[i].reshape(1, _H), beta[i].reshape(1, _H),
                          Ue1[i, :_H], Ue1[i, _H:2 * _H])
    tpre = _sc_pair(a, bgath, src3, dst3)
    if i + 1 < _L:
      ea, e = _tc_upd(ea, tpre, Ue1[i, 2 * _H:], ce1[i].reshape(1, _H),
                      Ue2[i], ce2[i].reshape(1, _H),
                      We[i + 1], be[i + 1].reshape(1, _H))
    else:
      ea = _tc_upd(ea, tpre, Ue1[i, 2 * _H:], ce1[i].reshape(1, _H),
                   Ue2[i], ce2[i].reshape(1, _H))
  return x, ea
